# baseline (speedup 1.0000x reference)
import functools

import jax
import jax.numpy as jnp
from jax import lax
from jax.experimental import pallas as pl
from jax.experimental.pallas import tpu as pltpu
from jax.experimental.pallas import tpu_sc as plsc

_NC, _NS = 2, 16
_NW = _NC * _NS
_IDXW = 128
_FUSE = 4
_REC = 128 * _FUSE


@functools.partial(jax.jit, static_argnums=(2,))
def _sc_lookup(table, idx2d, n_chunks):
    n_rows = idx2d.shape[0] * idx2d.shape[1]
    mesh = plsc.VectorSubcoreMesh(core_axis_name="c", subcore_axis_name="s")

    @functools.partial(
        pl.kernel,
        mesh=mesh,
        out_type=jax.ShapeDtypeStruct((n_rows, _REC), jnp.float32),
        scratch_types=[
            pltpu.VMEM((1, _IDXW), jnp.int32),
            pltpu.VMEM((_IDXW, _REC), jnp.float32),
            pltpu.SemaphoreType.DMA,
        ],
    )
    def k(table_hbm, idx_hbm, out_hbm, idx_v, rows_v, sem):
        wid = lax.axis_index("s") * _NC + lax.axis_index("c")
        row_base = wid * n_chunks

        def body(i, carry):
            r = row_base + i
            pltpu.sync_copy(idx_hbm.at[pl.ds(r, 1)], idx_v)
            pltpu.async_copy(table_hbm.at[idx_v.at[0]], rows_v, sem).wait()
            pltpu.sync_copy(rows_v, out_hbm.at[pl.ds(r * _IDXW, _IDXW)])
            return carry

        lax.fori_loop(0, n_chunks, body, 0)

    return k(table, idx2d)


def _quad_table(t):
    a = jnp.repeat(t, 64, axis=0)
    b = jnp.tile(jnp.repeat(t, 16, axis=0), (4, 1))
    c = jnp.tile(jnp.repeat(t, 4, axis=0), (16, 1))
    d = jnp.tile(t, (64, 1))
    return jnp.concatenate([a, b, c, d], axis=1)


def kernel(points, labels, point_embeddings, not_a_point_embed):
    b, n = labels.shape
    tot = b * n
    nrec = tot // _FUSE
    q = labels.reshape(nrec, _FUSE)
    idx = q[:, 0] * 64 + q[:, 1] * 16 + q[:, 2] * 4 + q[:, 3]
    idx2d = idx.reshape(nrec // _IDXW, _IDXW)
    n_chunks = nrec // (_NW * _IDXW)
    out = _sc_lookup(_quad_table(point_embeddings), idx2d, n_chunks)
    return out.reshape(b, n, 128)


# speedup vs baseline: 1.1372x; 1.1372x over previous
import functools

import jax
import jax.numpy as jnp
from jax import lax
from jax.experimental import pallas as pl
from jax.experimental.pallas import tpu as pltpu
from jax.experimental.pallas import tpu_sc as plsc

_EMBED = 128
_NC, _NS = 2, 16
_NW = _NC * _NS
_CHUNK = 128
_NBUF = 4
_REP = 32


@functools.partial(jax.jit, static_argnums=(2,))
def _sc_lookup(table, idx2d, n_chunks):
    n_rows = idx2d.shape[0] * idx2d.shape[1]
    n_quads = n_chunks // _NBUF
    mesh = plsc.VectorSubcoreMesh(core_axis_name="c", subcore_axis_name="s")

    @functools.partial(
        pl.kernel,
        mesh=mesh,
        out_type=jax.ShapeDtypeStruct((n_rows, _EMBED), jnp.float32),
        scratch_types=[
            pltpu.VMEM((n_chunks, _CHUNK), jnp.int32),
            pltpu.VMEM((_NBUF, _CHUNK, _EMBED), jnp.float32),
            pltpu.SemaphoreType.DMA((_NBUF,)),
            pltpu.SemaphoreType.DMA((_NBUF,)),
        ],
    )
    def k(table_hbm, idx_hbm, out_hbm, idx_v, rows_v, sem_g, sem_o):
        wid = lax.axis_index("s") * _NC + lax.axis_index("c")
        chunk0 = wid * n_chunks

        def fire_gather(c, s):
            pltpu.async_copy(table_hbm.at[idx_v.at[c]], rows_v.at[s],
                             sem_g.at[s])

        def wait_gather(s):
            pltpu.make_async_copy(out_hbm.at[pl.ds(0, _CHUNK)],
                                  rows_v.at[s], sem_g.at[s]).wait()

        def fire_out(c, s):
            pltpu.async_copy(rows_v.at[s],
                             out_hbm.at[pl.ds((chunk0 + c) * _CHUNK, _CHUNK)],
                             sem_o.at[s])

        def wait_out(s):
            pltpu.make_async_copy(rows_v.at[s],
                                  out_hbm.at[pl.ds(0, _CHUNK)],
                                  sem_o.at[s]).wait()

        pltpu.sync_copy(idx_hbm.at[pl.ds(chunk0, n_chunks)], idx_v)
        for s in range(_NBUF):
            fire_gather(s, s)

        def body(q, carry):
            c = q * _NBUF
            for s in range(_NBUF):
                wait_gather(s)
                fire_out(c + s, s)
            for s in range(_NBUF):
                wait_out(s)
                fire_gather(c + _NBUF + s, s)
            return carry

        lax.fori_loop(0, n_quads - 1, body, 0)

        c = (n_quads - 1) * _NBUF
        for s in range(_NBUF):
            wait_gather(s)
            fire_out(c + s, s)
        for s in range(_NBUF):
            wait_out(s)

    return k(table, idx2d)


def kernel(points, labels, point_embeddings, not_a_point_embed):
    b, n = labels.shape
    tot = b * n
    table_rep = jnp.tile(point_embeddings, (_REP, 1))
    flat = labels.reshape(tot)
    rep = (jnp.arange(tot, dtype=jnp.int32) % _REP) * 4
    idx2d = (flat + rep).reshape(tot // _CHUNK, _CHUNK)
    n_chunks = tot // (_NW * _CHUNK)
    out = _sc_lookup(table_rep, idx2d, n_chunks)
    return out.reshape(b, n, _EMBED)


# speedup vs baseline: 6.5182x; 5.7320x over previous
import functools

import jax
import jax.numpy as jnp
from jax import lax
from jax.experimental import pallas as pl
from jax.experimental.pallas import tpu as pltpu
from jax.experimental.pallas import tpu_sc as plsc

_EMBED = 128
_NC, _NS = 2, 16
_NW = _NC * _NS
_CHUNK = 128
_NBUF = 4
_UNROLL = 4


@functools.partial(jax.jit, static_argnums=(2,))
def _sc_lookup(table, idx2d, n_chunks):
    n_rows = idx2d.shape[0] * idx2d.shape[1]
    n_quads = n_chunks // _NBUF
    per_w = n_chunks * _CHUNK
    mesh = plsc.VectorSubcoreMesh(core_axis_name="c", subcore_axis_name="s")

    @functools.partial(
        pl.kernel,
        mesh=mesh,
        out_type=jax.ShapeDtypeStruct((n_rows, _EMBED), jnp.float32),
        scratch_types=[
            pltpu.VMEM((n_chunks, _CHUNK), jnp.int32),
            pltpu.VMEM((4, _EMBED), jnp.float32),
            pltpu.VMEM((_NBUF, _CHUNK, _EMBED), jnp.float32),
            pltpu.SemaphoreType.DMA((_NBUF,)),
        ],
    )
    def k(table_hbm, idx_hbm, out_hbm, idx_v, tab_v, rows_v, sem_o):
        wid = lax.axis_index("s") * _NC + lax.axis_index("c")
        chunk0 = wid * n_chunks

        def fire_out(c, s):
            pltpu.async_copy(rows_v.at[s],
                             out_hbm.at[pl.ds((chunk0 + c) * _CHUNK, _CHUNK)],
                             sem_o.at[s])

        def wait_out(s):
            pltpu.make_async_copy(rows_v.at[s],
                                  out_hbm.at[pl.ds(0, _CHUNK)],
                                  sem_o.at[s]).wait()

        pltpu.sync_copy(idx_hbm.at[pl.ds(chunk0, n_chunks)], idx_v)
        pltpu.sync_copy(table_hbm, tab_v)

        def make_compute_chunk():
            tv = [[tab_v[l, pl.ds(cc * 16, 16)] for cc in range(_EMBED // 16)]
                  for l in range(4)]

            def compute_chunk(c, s):
                def group_body(g, carry):
                    lblv = idx_v[c, pl.ds(g * 16, 16)]
                    for u in range(16):
                        r = g * 16 + u
                        lbl = lblv[u]
                        lo = lbl < 2
                        e0 = lbl == 0
                        e2 = lbl == 2
                        for cc in range(_EMBED // 16):
                            val = jnp.where(
                                lo,
                                jnp.where(e0, tv[0][cc], tv[1][cc]),
                                jnp.where(e2, tv[2][cc], tv[3][cc]))
                            rows_v[s, r, pl.ds(cc * 16, 16)] = val
                    return carry
                lax.fori_loop(0, _CHUNK // 16, group_body, 0)

            return compute_chunk

        compute_chunk = make_compute_chunk()

        for s in range(_NBUF):
            compute_chunk(s, s)
            fire_out(s, s)

        def body(q, carry):
            c0 = (q + 1) * _NBUF
            for s in range(_NBUF):
                wait_out(s)
                compute_chunk(c0 + s, s)
                fire_out(c0 + s, s)
            return carry

        lax.fori_loop(0, n_quads - 1, body, 0)
        for s in range(_NBUF):
            wait_out(s)

    return k(table, idx2d)


def kernel(points, labels, point_embeddings, not_a_point_embed):
    b, n = labels.shape
    tot = b * n
    idx2d = labels.reshape(tot // _CHUNK, _CHUNK)
    n_chunks = tot // (_NW * _CHUNK)
    out = _sc_lookup(point_embeddings, idx2d, n_chunks)
    return out.reshape(b, n, _EMBED)
